# baseline (speedup 1.0000x reference)
import jax
import jax.numpy as jnp
from jax.experimental import pallas as pl
from jax.experimental.pallas import tpu as pltpu

_NQ = 8
_K = 1024
_D = 256
_BLK = 512
_H = _BLK // 2


def _rvq_block(z_ref, cb_ref, qsum_ref, idx_ref,
               c1_ref, c2_ref, c3_ref, aux_ref, oh_ref, idxs_ref):
    @pl.when(pl.program_id(0) == 0)
    def _split():
        cb = cb_ref[...]
        c1 = cb.astype(jnp.bfloat16)
        e1 = cb - c1.astype(jnp.float32)
        c2 = e1.astype(jnp.bfloat16)
        e2 = e1 - c2.astype(jnp.float32)
        c1_ref[...] = c1
        c2_ref[...] = c2
        c3_ref[...] = e2.astype(jnp.bfloat16)
        k_i = jax.lax.broadcasted_iota(jnp.int32, (_K, 128), 0)
        lane = jax.lax.broadcasted_iota(jnp.int32, (_K, 128), 1)
        vals = jnp.where(lane == 0, 1.0,
                         jnp.where(lane == 1, (k_i >> 6).astype(jnp.float32),
                                   jnp.where(lane == 2,
                                             (k_i & 63).astype(jnp.float32),
                                             0.0)))
        aux_ref[...] = vals.astype(jnp.bfloat16)

    dims = (((1,), (0,)), ((), ()))
    rs = [z_ref[:_H, :], z_ref[_H:, :]]
    qsums = [jnp.zeros_like(rs[0]), jnp.zeros_like(rs[1])]
    for i in range(_NQ):
        cb = cb_ref[i]
        c2 = jnp.sum(cb * cb, axis=1)[None, :]
        last = i == _NQ - 1
        for h in range(2):
            r = rs[h]
            sl = slice(h * _H, (h + 1) * _H)
            r2 = jnp.sum(r * r, axis=1, keepdims=True)
            rc = jax.lax.dot_general(r, cb, (((1,), (1,)), ((), ())),
                                     preferred_element_type=jnp.float32)
            d = jnp.sqrt(jnp.maximum(r2 + c2 - 2.0 * rc, 0.0))
            m = jnp.min(d, axis=1, keepdims=True)
            oh0 = (d == m).astype(jnp.bfloat16)
            aug = jax.lax.dot_general(oh0, aux_ref[...], dims,
                                      preferred_element_type=jnp.float32)
            cnt = aug[:, 0:1]
            oh_ref[sl, :] = oh0
            if last:
                idxf = aug[:, 1:2] * 64.0 + aug[:, 2:3]
                idxs_ref[sl, :] = idxf.astype(jnp.int32)

            @pl.when(jnp.any(cnt != 1.0))
            def _fix(d=d, sl=sl, last=last):
                idx = jnp.argmin(d, axis=1)
                oh_ref[sl, :] = (
                    jax.lax.broadcasted_iota(jnp.int32, (_H, _K), 1)
                    == idx[:, None]).astype(jnp.bfloat16)
                if last:
                    idxs_ref[sl, :] = idx[:, None].astype(jnp.int32)

            oh = oh_ref[sl, :]
            q = jax.lax.dot_general(oh, c3_ref[i], dims,
                                    preferred_element_type=jnp.float32)
            q = q + jax.lax.dot_general(oh, c2_ref[i], dims,
                                        preferred_element_type=jnp.float32)
            q = q + jax.lax.dot_general(oh, c1_ref[i], dims,
                                        preferred_element_type=jnp.float32)
            qsums[h] = qsums[h] + q
            rs[h] = r - q
    qsum_ref[:_H, :] = qsums[0]
    qsum_ref[_H:, :] = qsums[1]
    idx_ref[...] = idxs_ref[...]


def kernel(z, codebooks):
    batch = z.shape[0]
    qsum, idx = pl.pallas_call(
        _rvq_block,
        grid=(batch // _BLK,),
        in_specs=[
            pl.BlockSpec((_BLK, _D), lambda i: (i, 0)),
            pl.BlockSpec((_NQ, _K, _D), lambda i: (0, 0, 0)),
        ],
        out_specs=[
            pl.BlockSpec((_BLK, _D), lambda i: (i, 0)),
            pl.BlockSpec((_BLK, 1), lambda i: (i, 0)),
        ],
        out_shape=[
            jax.ShapeDtypeStruct((batch, _D), jnp.float32),
            jax.ShapeDtypeStruct((batch, 1), jnp.int32),
        ],
        scratch_shapes=[
            pltpu.VMEM((_NQ, _K, _D), jnp.bfloat16),
            pltpu.VMEM((_NQ, _K, _D), jnp.bfloat16),
            pltpu.VMEM((_NQ, _K, _D), jnp.bfloat16),
            pltpu.VMEM((_K, 128), jnp.bfloat16),
            pltpu.VMEM((_BLK, _K), jnp.bfloat16),
            pltpu.VMEM((_BLK, 1), jnp.int32),
        ],
    )(z, codebooks)
    return (qsum, idx)


# speedup vs baseline: 1.2696x; 1.2696x over previous
import jax
import jax.numpy as jnp
from jax.experimental import pallas as pl
from jax.experimental.pallas import tpu as pltpu

_NQ = 8
_K = 1024
_D = 256
_BLK = 1024
_H = _BLK // 2


def _rvq_block(z_ref, cb_ref, qsum_ref, idx_ref, c1_ref, c2_ref, c3_ref):
    @pl.when(pl.program_id(0) == 0)
    def _split():
        cb = cb_ref[...]
        c1 = cb.astype(jnp.bfloat16)
        e1 = cb - c1.astype(jnp.float32)
        c2 = e1.astype(jnp.bfloat16)
        e2 = e1 - c2.astype(jnp.float32)
        c1_ref[...] = c1
        c2_ref[...] = c2
        c3_ref[...] = e2.astype(jnp.bfloat16)

    rs = [z_ref[:_H, :], z_ref[_H:, :]]
    qsums = [jnp.zeros_like(rs[0]), jnp.zeros_like(rs[1])]
    idxs = [None, None]
    for i in range(_NQ):
        cb = cb_ref[i]
        c2 = jnp.sum(cb * cb, axis=1)[None, :]
        for h in range(2):
            r = rs[h]
            r2 = jnp.sum(r * r, axis=1, keepdims=True)
            rc = jax.lax.dot_general(r, cb, (((1,), (1,)), ((), ())),
                                     preferred_element_type=jnp.float32)
            d = jnp.sqrt(jnp.maximum(r2 + c2 - 2.0 * rc, 0.0))
            idx = jnp.argmin(d, axis=1)
            oh = (jax.lax.broadcasted_iota(jnp.int32, (_H, _K), 1)
                  == idx[:, None]).astype(jnp.bfloat16)
            dims = (((1,), (0,)), ((), ()))
            q = jax.lax.dot_general(oh, c3_ref[i], dims,
                                    preferred_element_type=jnp.float32)
            q = q + jax.lax.dot_general(oh, c2_ref[i], dims,
                                        preferred_element_type=jnp.float32)
            q = q + jax.lax.dot_general(oh, c1_ref[i], dims,
                                        preferred_element_type=jnp.float32)
            qsums[h] = qsums[h] + q
            rs[h] = r - q
            idxs[h] = idx
    qsum_ref[:_H, :] = qsums[0]
    qsum_ref[_H:, :] = qsums[1]
    idx_ref[:_H, :] = idxs[0][:, None].astype(jnp.int32)
    idx_ref[_H:, :] = idxs[1][:, None].astype(jnp.int32)


def kernel(z, codebooks):
    batch = z.shape[0]
    qsum, idx = pl.pallas_call(
        _rvq_block,
        grid=(batch // _BLK,),
        in_specs=[
            pl.BlockSpec((_BLK, _D), lambda i: (i, 0)),
            pl.BlockSpec((_NQ, _K, _D), lambda i: (0, 0, 0)),
        ],
        out_specs=[
            pl.BlockSpec((_BLK, _D), lambda i: (i, 0)),
            pl.BlockSpec((_BLK, 1), lambda i: (i, 0)),
        ],
        out_shape=[
            jax.ShapeDtypeStruct((batch, _D), jnp.float32),
            jax.ShapeDtypeStruct((batch, 1), jnp.int32),
        ],
        scratch_shapes=[
            pltpu.VMEM((_NQ, _K, _D), jnp.bfloat16),
            pltpu.VMEM((_NQ, _K, _D), jnp.bfloat16),
            pltpu.VMEM((_NQ, _K, _D), jnp.bfloat16),
        ],
    )(z, codebooks)
    return (qsum, idx)
